# baseline (speedup 1.0000x reference)
import jax
import jax.numpy as jnp
from jax import lax
from jax.experimental import pallas as pl
from jax.experimental.pallas import tpu as pltpu

_K = 8192
_D = 64
_TILE = 512
_KBLK = 4096
_COMMIT = 0.25


def _vq_body(x_ref, x2_ref, w_ref, w2_ref, wb_ref,
             recon_ref, loss_ref, perp_ref,
             hist_ref, acc_ref):
    step = pl.program_id(0)
    nsteps = pl.num_programs(0)

    @pl.when(step == 0)
    def _init():
        hist_ref[...] = jnp.zeros_like(hist_ref)
        acc_ref[...] = jnp.zeros_like(acc_ref)

    x = x_ref[...]
    x2 = x2_ref[...]

    carry_v = jnp.full((_TILE, 1), jnp.inf, jnp.float32)
    carry_i = jnp.zeros((_TILE, 1), jnp.int32)
    for b in range(_K // _KBLK):
        wb = w_ref[b * _KBLK:(b + 1) * _KBLK, :]
        w2b = w2_ref[:, b * _KBLK:(b + 1) * _KBLK]
        mm = lax.dot_general(x, wb, (((1,), (1,)), ((), ())),
                             preferred_element_type=jnp.float32)
        dist = (x2 + w2b) - 2.0 * mm
        vb = jnp.min(dist, axis=1, keepdims=True)
        iota = lax.broadcasted_iota(jnp.int32, (_TILE, _KBLK), 1)
        ib = jnp.min(jnp.where(dist == vb, iota, _KBLK),
                     axis=1, keepdims=True) + b * _KBLK
        lt = vb < carry_v
        upd = lt | ((vb == carry_v) & (ib < carry_i))
        carry_i = jnp.where(upd, ib, carry_i)
        carry_v = jnp.where(lt, vb, carry_v)
        carry_v = carry_v.astype(jnp.bfloat16).astype(jnp.float32)

    iota_full = lax.broadcasted_iota(jnp.int32, (_TILE, _K), 1)
    onehot = (iota_full == carry_i).astype(jnp.float32).astype(jnp.bfloat16)
    q = jnp.dot(onehot, wb_ref[...],
                preferred_element_type=jnp.float32)
    recon_ref[...] = x + (q - x)
    ones_row = jnp.full((1, _TILE), jnp.bfloat16(1))
    hist_ref[...] += jnp.dot(ones_row, onehot,
                             preferred_element_type=jnp.float32)
    d = q - x
    acc_ref[...] += jnp.sum(d * d).reshape(1, 1)

    @pl.when(step == nsteps - 1)
    def _finish():
        n = nsteps * _TILE
        avg = hist_ref[...] / jnp.float32(n)
        perp = jnp.exp(-jnp.sum(avg * jnp.log(avg + 1e-10)))
        perp_ref[...] = perp.reshape(1, 1)
        m = acc_ref[0, 0] / jnp.float32(n * _D)
        loss_ref[...] = (m + _COMMIT * m).reshape(1, 1)


def kernel(x, W):
    flat = x.reshape(-1, _D)
    n = flat.shape[0]
    x2 = jnp.sum(flat ** 2, axis=1, keepdims=True)
    w2 = jnp.sum(W ** 2, axis=1)[None, :]
    wb = W.astype(jnp.bfloat16)
    grid = n // _TILE
    recon, loss, perp = pl.pallas_call(
        _vq_body,
        grid=(grid,),
        in_specs=[
            pl.BlockSpec((_TILE, _D), lambda i: (i, 0)),
            pl.BlockSpec((_TILE, 1), lambda i: (i, 0)),
            pl.BlockSpec((_K, _D), lambda i: (0, 0)),
            pl.BlockSpec((1, _K), lambda i: (0, 0)),
            pl.BlockSpec((_K, _D), lambda i: (0, 0)),
        ],
        out_specs=[
            pl.BlockSpec((_TILE, _D), lambda i: (i, 0)),
            pl.BlockSpec((1, 1), lambda i: (0, 0)),
            pl.BlockSpec((1, 1), lambda i: (0, 0)),
        ],
        out_shape=[
            jax.ShapeDtypeStruct((n, _D), jnp.float32),
            jax.ShapeDtypeStruct((1, 1), jnp.float32),
            jax.ShapeDtypeStruct((1, 1), jnp.float32),
        ],
        scratch_shapes=[
            pltpu.VMEM((1, _K), jnp.float32),
            pltpu.VMEM((1, 1), jnp.float32),
        ],
    )(flat, x2, W, w2, wb)
    return loss[0, 0], recon.reshape(x.shape), perp[0, 0]


# speedup vs baseline: 1.1241x; 1.1241x over previous
import jax
import jax.numpy as jnp
from jax import lax
from jax.experimental import pallas as pl
from jax.experimental.pallas import tpu as pltpu

_K = 8192
_D = 64
_TILE = 256
_KBLK = 4096
_COMMIT = 0.25


def _vq_body(x_ref, x2_ref, w_ref, w2_ref, wb_ref,
             recon_ref, loss_ref, perp_ref,
             hist_ref, acc_ref):
    step = pl.program_id(0)
    nsteps = pl.num_programs(0)

    @pl.when(step == 0)
    def _init():
        hist_ref[...] = jnp.zeros_like(hist_ref)
        acc_ref[...] = jnp.zeros_like(acc_ref)

    x = x_ref[...]
    x2 = x2_ref[...]

    carry_v = jnp.full((_TILE, 1), jnp.inf, jnp.float32)
    carry_i = jnp.zeros((_TILE, 1), jnp.int32)
    for b in range(_K // _KBLK):
        wb = w_ref[b * _KBLK:(b + 1) * _KBLK, :]
        w2b = w2_ref[:, b * _KBLK:(b + 1) * _KBLK]
        mm = lax.dot_general(x, wb, (((1,), (1,)), ((), ())),
                             preferred_element_type=jnp.float32)
        dist = (x2 + w2b) + mm
        vb = jnp.min(dist, axis=1, keepdims=True)
        iota = lax.broadcasted_iota(jnp.int32, (_TILE, _KBLK), 1)
        ib = jnp.min(jnp.where(dist == vb, iota, _KBLK),
                     axis=1, keepdims=True) + b * _KBLK
        lt = vb < carry_v
        upd = lt | ((vb == carry_v) & (ib < carry_i))
        carry_i = jnp.where(upd, ib, carry_i)
        carry_v = jnp.where(lt, vb, carry_v)
        carry_v = carry_v.astype(jnp.bfloat16).astype(jnp.float32)

    iota_full = lax.broadcasted_iota(jnp.int32, (_TILE, _K), 1)
    onehot = (iota_full == carry_i).astype(jnp.float32).astype(jnp.bfloat16)
    q = jnp.dot(onehot, wb_ref[...],
                preferred_element_type=jnp.float32)
    recon_ref[...] = x + (q - x)
    ones_row = jnp.full((1, _TILE), jnp.bfloat16(1))
    hist_ref[...] += jnp.dot(ones_row, onehot,
                             preferred_element_type=jnp.float32)
    d = q - x
    acc_ref[...] += jnp.sum(d * d).reshape(1, 1)

    @pl.when(step == nsteps - 1)
    def _finish():
        n = nsteps * _TILE
        avg = hist_ref[...] / jnp.float32(n)
        perp = jnp.exp(-jnp.sum(avg * jnp.log(avg + 1e-10)))
        perp_ref[...] = perp.reshape(1, 1)
        m = acc_ref[0, 0] / jnp.float32(n * _D)
        loss_ref[...] = (m + _COMMIT * m).reshape(1, 1)


def kernel(x, W):
    flat = x.reshape(-1, _D)
    n = flat.shape[0]
    x2 = jnp.sum(flat ** 2, axis=1, keepdims=True)
    w2 = jnp.sum(W ** 2, axis=1)[None, :]
    w2n = -2.0 * W
    wb = W.astype(jnp.bfloat16)
    grid = n // _TILE
    recon, loss, perp = pl.pallas_call(
        _vq_body,
        grid=(grid,),
        in_specs=[
            pl.BlockSpec((_TILE, _D), lambda i: (i, 0)),
            pl.BlockSpec((_TILE, 1), lambda i: (i, 0)),
            pl.BlockSpec((_K, _D), lambda i: (0, 0)),
            pl.BlockSpec((1, _K), lambda i: (0, 0)),
            pl.BlockSpec((_K, _D), lambda i: (0, 0)),
        ],
        out_specs=[
            pl.BlockSpec((_TILE, _D), lambda i: (i, 0)),
            pl.BlockSpec((1, 1), lambda i: (0, 0)),
            pl.BlockSpec((1, 1), lambda i: (0, 0)),
        ],
        out_shape=[
            jax.ShapeDtypeStruct((n, _D), jnp.float32),
            jax.ShapeDtypeStruct((1, 1), jnp.float32),
            jax.ShapeDtypeStruct((1, 1), jnp.float32),
        ],
        scratch_shapes=[
            pltpu.VMEM((1, _K), jnp.float32),
            pltpu.VMEM((1, 1), jnp.float32),
        ],
    )(flat, x2, w2n, w2, wb)
    return loss[0, 0], recon.reshape(x.shape), perp[0, 0]


# speedup vs baseline: 1.1662x; 1.0375x over previous
import jax
import jax.numpy as jnp
from jax import lax
from jax.experimental import pallas as pl
from jax.experimental.pallas import tpu as pltpu

_K = 8192
_D = 64
_TILE = 256
_KBLK = 4096
_COMMIT = 0.25


def _vq_body(x_ref, x2_ref, w_ref, w2_ref, wb_ref,
             recon_ref, loss_ref, perp_ref,
             hist_ref, acc_ref):
    step = pl.program_id(0)
    nsteps = pl.num_programs(0)

    @pl.when(step == 0)
    def _init():
        hist_ref[...] = jnp.zeros_like(hist_ref)
        acc_ref[...] = jnp.zeros_like(acc_ref)

    x = x_ref[...]
    x2 = x2_ref[...]

    carry_v = jnp.full((_TILE, 1), jnp.inf, jnp.float32)
    carry_i = jnp.zeros((_TILE, 1), jnp.int32)
    for b in range(_K // _KBLK):
        wb = w_ref[b * _KBLK:(b + 1) * _KBLK, :]
        w2b = w2_ref[:, b * _KBLK:(b + 1) * _KBLK]
        mm = lax.dot_general(x, wb, (((1,), (1,)), ((), ())),
                             preferred_element_type=jnp.float32)
        dist = (x2 + w2b) + mm
        vb = jnp.min(dist, axis=1, keepdims=True)
        iota = lax.broadcasted_iota(jnp.int32, (_TILE, _KBLK), 1)
        ib = jnp.min(jnp.where(dist == vb, iota, _KBLK),
                     axis=1, keepdims=True) + b * _KBLK
        lt = vb < carry_v
        upd = lt | ((vb == carry_v) & (ib < carry_i))
        carry_i = jnp.where(upd, ib, carry_i)
        carry_v = jnp.where(lt, vb, carry_v)
        carry_v = carry_v.astype(jnp.bfloat16).astype(jnp.float32)

    iota_full = lax.broadcasted_iota(jnp.int32, (_TILE, _K), 1)
    onehot = (iota_full == carry_i).astype(jnp.float32)
    q = jnp.dot(onehot.astype(jnp.bfloat16), wb_ref[...],
                preferred_element_type=jnp.float32)
    recon_ref[...] = x + (q - x)
    hist_ref[...] += jnp.sum(onehot, axis=0, keepdims=True)
    d = q - x
    acc_ref[...] += jnp.sum(d * d).reshape(1, 1)

    @pl.when(step == nsteps - 1)
    def _finish():
        n = nsteps * _TILE
        avg = hist_ref[...] / jnp.float32(n)
        perp = jnp.exp(-jnp.sum(avg * jnp.log(avg + 1e-10)))
        perp_ref[...] = perp.reshape(1, 1)
        m = acc_ref[0, 0] / jnp.float32(n * _D)
        loss_ref[...] = (m + _COMMIT * m).reshape(1, 1)


def kernel(x, W):
    flat = x.reshape(-1, _D)
    n = flat.shape[0]
    x2 = jnp.sum(flat ** 2, axis=1, keepdims=True)
    w2 = jnp.sum(W ** 2, axis=1)[None, :]
    w2n = -2.0 * W
    wb = W.astype(jnp.bfloat16)
    grid = n // _TILE
    recon, loss, perp = pl.pallas_call(
        _vq_body,
        grid=(grid,),
        in_specs=[
            pl.BlockSpec((_TILE, _D), lambda i: (i, 0)),
            pl.BlockSpec((_TILE, 1), lambda i: (i, 0)),
            pl.BlockSpec((_K, _D), lambda i: (0, 0)),
            pl.BlockSpec((1, _K), lambda i: (0, 0)),
            pl.BlockSpec((_K, _D), lambda i: (0, 0)),
        ],
        out_specs=[
            pl.BlockSpec((_TILE, _D), lambda i: (i, 0)),
            pl.BlockSpec((1, 1), lambda i: (0, 0)),
            pl.BlockSpec((1, 1), lambda i: (0, 0)),
        ],
        out_shape=[
            jax.ShapeDtypeStruct((n, _D), jnp.float32),
            jax.ShapeDtypeStruct((1, 1), jnp.float32),
            jax.ShapeDtypeStruct((1, 1), jnp.float32),
        ],
        scratch_shapes=[
            pltpu.VMEM((1, _K), jnp.float32),
            pltpu.VMEM((1, 1), jnp.float32),
        ],
    )(flat, x2, w2n, w2, wb)
    return loss[0, 0], recon.reshape(x.shape), perp[0, 0]


# speedup vs baseline: 1.2718x; 1.0905x over previous
import jax
import jax.numpy as jnp
from jax import lax
from jax.experimental import pallas as pl
from jax.experimental.pallas import tpu as pltpu

_K = 8192
_D = 64
_TILE = 256
_KBLK = 4096
_COMMIT = 0.25


def _vq_body(x_ref, x2_ref, w_ref, w2_ref, wb_ref,
             recon_ref, loss_ref, perp_ref,
             hist_ref, acc_ref):
    step = pl.program_id(0)
    nsteps = pl.num_programs(0)

    @pl.when(step == 0)
    def _init():
        hist_ref[...] = jnp.zeros_like(hist_ref)
        acc_ref[...] = jnp.zeros_like(acc_ref)

    x = x_ref[...]
    x2 = x2_ref[...]

    carry_v = jnp.full((_TILE, 1), jnp.inf, jnp.float32)
    carry_i = jnp.zeros((_TILE, 1), jnp.int32)
    for b in range(_K // _KBLK):
        wb = w_ref[b * _KBLK:(b + 1) * _KBLK, :]
        w2b = w2_ref[:, b * _KBLK:(b + 1) * _KBLK]
        mm = lax.dot_general(x, wb, (((1,), (1,)), ((), ())),
                             preferred_element_type=jnp.float32)
        dist = (x2 + w2b) - 2.0 * mm
        vb = jnp.min(dist, axis=1, keepdims=True)
        iota = lax.broadcasted_iota(jnp.int32, (_TILE, _KBLK), 1)
        ib = jnp.min(jnp.where(dist == vb, iota, _KBLK),
                     axis=1, keepdims=True) + b * _KBLK
        lt = vb < carry_v
        upd = lt | ((vb == carry_v) & (ib < carry_i))
        carry_i = jnp.where(upd, ib, carry_i)
        carry_v = jnp.where(lt, vb, carry_v)
        carry_v = carry_v.astype(jnp.bfloat16).astype(jnp.float32)

    iota_full = lax.broadcasted_iota(jnp.int32, (_TILE, _K), 1)
    onehot = (iota_full == carry_i).astype(jnp.float32)
    q = jnp.dot(onehot.astype(jnp.bfloat16), wb_ref[...],
                preferred_element_type=jnp.float32)
    recon_ref[...] = x + (q - x)
    hist_ref[...] += jnp.sum(onehot, axis=0, keepdims=True)
    d = q - x
    acc_ref[...] += jnp.sum(d * d).reshape(1, 1)

    @pl.when(step == nsteps - 1)
    def _finish():
        n = nsteps * _TILE
        avg = hist_ref[...] / jnp.float32(n)
        perp = jnp.exp(-jnp.sum(avg * jnp.log(avg + 1e-10)))
        perp_ref[...] = perp.reshape(1, 1)
        m = acc_ref[0, 0] / jnp.float32(n * _D)
        loss_ref[...] = (m + _COMMIT * m).reshape(1, 1)


def kernel(x, W):
    flat = x.reshape(-1, _D)
    n = flat.shape[0]
    x2 = jnp.sum(flat ** 2, axis=1, keepdims=True)
    w2 = jnp.sum(W ** 2, axis=1)[None, :]
    wb = W.astype(jnp.bfloat16)
    grid = n // _TILE
    recon, loss, perp = pl.pallas_call(
        _vq_body,
        grid=(grid,),
        in_specs=[
            pl.BlockSpec((_TILE, _D), lambda i: (i, 0)),
            pl.BlockSpec((_TILE, 1), lambda i: (i, 0)),
            pl.BlockSpec((_K, _D), lambda i: (0, 0)),
            pl.BlockSpec((1, _K), lambda i: (0, 0)),
            pl.BlockSpec((_K, _D), lambda i: (0, 0)),
        ],
        out_specs=[
            pl.BlockSpec((_TILE, _D), lambda i: (i, 0)),
            pl.BlockSpec((1, 1), lambda i: (0, 0)),
            pl.BlockSpec((1, 1), lambda i: (0, 0)),
        ],
        out_shape=[
            jax.ShapeDtypeStruct((n, _D), jnp.float32),
            jax.ShapeDtypeStruct((1, 1), jnp.float32),
            jax.ShapeDtypeStruct((1, 1), jnp.float32),
        ],
        scratch_shapes=[
            pltpu.VMEM((1, _K), jnp.float32),
            pltpu.VMEM((1, 1), jnp.float32),
        ],
    )(flat, x2, W, w2, wb)
    return loss[0, 0], recon.reshape(x.shape), perp[0, 0]
